# baseline (speedup 1.0000x reference)
import functools

import jax
import jax.numpy as jnp
from jax import lax
from jax.experimental import pallas as pl
from jax.experimental.pallas import tpu as pltpu
from jax.experimental.pallas import tpu_sc as plsc

N, E, D, DE, H = 10000, 320000, 128, 4, 128
NPAD = 10240
NC, NS = 2, 16
NW = NC * NS
PER_W = E // NW
C = 80
CH = PER_W // C
ROWS_PER_TILE = NPAD // NS

f32 = jnp.float32
i32 = jnp.int32


def _prep_body(h_ref, w1a_ref, w1b_ref, ta_ref, tb_ref):
    hb = h_ref[...]
    ta_ref[...] = jnp.dot(hb, w1a_ref[...], preferred_element_type=f32)
    tb_ref[...] = jnp.dot(hb, w1b_ref[...], preferred_element_type=f32)


def _prep_tables(h, w1a, w1b):
    bn = 1000
    return pl.pallas_call(
        _prep_body,
        grid=(N // bn,),
        in_specs=[
            pl.BlockSpec((bn, 128), lambda i: (i, 0)),
            pl.BlockSpec((128, 128), lambda i: (0, 0)),
            pl.BlockSpec((128, 128), lambda i: (0, 0)),
        ],
        out_specs=[
            pl.BlockSpec((bn, 128), lambda i: (i, 0)),
            pl.BlockSpec((bn, 128), lambda i: (i, 0)),
        ],
        out_shape=[
            jax.ShapeDtypeStruct((N, 128), f32),
            jax.ShapeDtypeStruct((N, 128), f32),
        ],
    )(h, w1a, w1b)


def _gather_pre_body(ta, tb, row, col, out, idxr, idxc, bufr, bufc,
                     semr, semc):
    c = lax.axis_index("c")
    s = lax.axis_index("s")
    wid = s * NC + c
    base = wid * PER_W

    def chunk(k, carry):
        b = base + k * C
        pltpu.sync_copy(row.at[pl.ds(b, C)], idxr)
        pltpu.sync_copy(col.at[pl.ds(b, C)], idxc)
        cp1 = pltpu.async_copy(ta.at[idxr], bufr, semr)
        cp2 = pltpu.async_copy(tb.at[idxc], bufc, semc)
        cp1.wait()
        cp2.wait()

        def rowfn(i, carry2):
            for j in range(8):
                sl = pl.ds(j * 16, 16)
                bufr[i, sl] = bufr[i, sl] + bufc[i, sl]
            return carry2

        lax.fori_loop(0, C, rowfn, 0)
        pltpu.sync_copy(bufr, out.at[pl.ds(b, C)])
        return carry

    lax.fori_loop(0, CH, chunk, 0)


@functools.cache
def _gather_pre():
    return pl.kernel(
        _gather_pre_body,
        out_type=jax.ShapeDtypeStruct((E, 128), f32),
        mesh=plsc.VectorSubcoreMesh(core_axis_name="c", subcore_axis_name="s",
                                    num_cores=NC, num_subcores=NS),
        scratch_types=[
            pltpu.VMEM((C,), i32),
            pltpu.VMEM((C,), i32),
            pltpu.VMEM((C, 128), f32),
            pltpu.VMEM((C, 128), f32),
            pltpu.SemaphoreType.DMA,
            pltpu.SemaphoreType.DMA,
        ],
    )


def _coord_feats_body(c16, row, col, out, idxr, idxc, bufr, bufc,
                      semr, semc):
    c = lax.axis_index("c")
    s = lax.axis_index("s")
    wid = s * NC + c
    base = wid * PER_W
    lane = lax.broadcasted_iota(i32, (16,), 0)

    def chunk(k, carry):
        b = base + k * C
        pltpu.sync_copy(row.at[pl.ds(b, C)], idxr)
        pltpu.sync_copy(col.at[pl.ds(b, C)], idxc)
        cp1 = pltpu.async_copy(c16.at[idxr], bufr, semr)
        cp2 = pltpu.async_copy(c16.at[idxc], bufc, semc)
        cp1.wait()
        cp2.wait()

        def rowfn(i, carry2):
            d = bufr[i, :] - bufc[i, :]
            rad = lax.reduce_sum(d * d, axes=(0,))
            bufr[i, :] = jnp.where(lane == 3, rad, d)
            return carry2

        lax.fori_loop(0, C, rowfn, 0)
        pltpu.sync_copy(bufr, out.at[pl.ds(b, C)])
        return carry

    lax.fori_loop(0, CH, chunk, 0)


@functools.cache
def _coord_feats():
    return pl.kernel(
        _coord_feats_body,
        out_type=jax.ShapeDtypeStruct((E, 16), f32),
        mesh=plsc.VectorSubcoreMesh(core_axis_name="c", subcore_axis_name="s",
                                    num_cores=NC, num_subcores=NS),
        compiler_params=pltpu.CompilerParams(use_tc_tiling_on_sc=False,
                                             needs_layout_passes=False),
        scratch_types=[
            pltpu.VMEM((C,), i32),
            pltpu.VMEM((C,), i32),
            pltpu.VMEM((C, 16), f32),
            pltpu.VMEM((C, 16), f32),
            pltpu.SemaphoreType.DMA,
            pltpu.SemaphoreType.DMA,
        ],
    )


def _edge_body(pre_ref, dr_ref, ea_ref, wr_ref, wea_ref, be1_ref,
               we2_ref, be2_ref, wc1_ref, bc1_ref, wc2_ref, bc2_ref,
               m_ref, t8_ref):
    dr = dr_ref[...]
    rad = dr[:, 3:4]
    x1 = (pre_ref[...] + rad * wr_ref[...] +
          jnp.dot(ea_ref[...], wea_ref[...], preferred_element_type=f32) +
          be1_ref[...])
    x1 = jnp.maximum(x1, 0.0)
    m = jnp.maximum(
        jnp.dot(x1, we2_ref[...], preferred_element_type=f32) + be2_ref[...],
        0.0)
    m_ref[...] = m
    cfh = jnp.maximum(
        jnp.dot(m, wc1_ref[...], preferred_element_type=f32) + bc1_ref[...],
        0.0)
    cf = jnp.dot(cfh, wc2_ref[...], preferred_element_type=f32) + bc2_ref[...]
    t = dr[:, 0:8] * cf
    iot = lax.broadcasted_iota(i32, t.shape, 1)
    t8_ref[...] = jnp.where(iot == 3, 1.0, t)


def _edge_mlp(pre, dr, edge_attr, wr, wea, be1, we2, be2, wc1, bc1, wc2, bc2):
    be = 2000
    wfull = lambda shape: pl.BlockSpec(shape, lambda i: (0, 0))
    return pl.pallas_call(
        _edge_body,
        grid=(E // be,),
        in_specs=[
            pl.BlockSpec((be, 128), lambda i: (i, 0)),
            pl.BlockSpec((be, 16), lambda i: (i, 0)),
            pl.BlockSpec((be, DE), lambda i: (i, 0)),
            wfull((1, 128)), wfull((DE, 128)), wfull((1, 128)),
            wfull((128, 128)), wfull((1, 128)),
            wfull((128, 128)), wfull((1, 128)),
            wfull((128, 1)), wfull((1, 1)),
        ],
        out_specs=[
            pl.BlockSpec((be, 128), lambda i: (i, 0)),
            pl.BlockSpec((be, 8), lambda i: (i, 0)),
        ],
        out_shape=[
            jax.ShapeDtypeStruct((E, 128), f32),
            jax.ShapeDtypeStruct((E, 8), f32),
        ],
    )(pre, dr, edge_attr, wr, wea, be1, we2, be2, wc1, bc1, wc2, bc2)


def _scatter_m_body(mij, row, z128, agg_out, idxv, mbuf, aggsh):
    c = lax.axis_index("c")
    s = lax.axis_index("s")
    wid = s * NC + c
    base = wid * PER_W
    rsl = pl.ds(s * ROWS_PER_TILE, ROWS_PER_TILE)
    pltpu.sync_copy(z128.at[rsl], aggsh.at[rsl])
    plsc.subcore_barrier()

    def chunk(k, carry):
        b = base + k * C
        pltpu.sync_copy(row.at[pl.ds(b, C)], idxv)
        pltpu.sync_copy(mij.at[pl.ds(b, C)], mbuf)
        pltpu.sync_copy(mbuf, aggsh.at[idxv], add=True)
        return carry

    lax.fori_loop(0, CH, chunk, 0)
    plsc.subcore_barrier()
    pltpu.sync_copy(aggsh.at[rsl], agg_out.at[c].at[rsl])


@functools.cache
def _scatter_m():
    return pl.kernel(
        _scatter_m_body,
        out_type=jax.ShapeDtypeStruct((NC, NPAD, 128), f32),
        mesh=plsc.VectorSubcoreMesh(core_axis_name="c", subcore_axis_name="s",
                                    num_cores=NC, num_subcores=NS),
        scratch_types=[
            pltpu.VMEM((C,), i32),
            pltpu.VMEM((C, 128), f32),
            pltpu.VMEM_SHARED((NPAD, 128), f32),
        ],
    )


def _scatter_t_body(t8, row, z16, t_out, idxv, tbuf, tsh):
    c = lax.axis_index("c")
    s = lax.axis_index("s")
    wid = s * NC + c
    base = wid * PER_W
    rsl = pl.ds(s * ROWS_PER_TILE, ROWS_PER_TILE)
    pltpu.sync_copy(z16.at[rsl], tsh.at[rsl])
    plsc.subcore_barrier()

    def chunk(k, carry):
        b = base + k * C
        pltpu.sync_copy(row.at[pl.ds(b, C)], idxv)
        pltpu.sync_copy(t8.at[pl.ds(b, C)], tbuf)
        pltpu.sync_copy(tbuf, tsh.at[idxv], add=True)
        return carry

    lax.fori_loop(0, CH, chunk, 0)
    plsc.subcore_barrier()
    pltpu.sync_copy(tsh.at[rsl], t_out.at[c].at[rsl])


@functools.cache
def _scatter_t():
    return pl.kernel(
        _scatter_t_body,
        out_type=jax.ShapeDtypeStruct((NC, NPAD, 8), f32),
        mesh=plsc.VectorSubcoreMesh(core_axis_name="c", subcore_axis_name="s",
                                    num_cores=NC, num_subcores=NS),
        compiler_params=pltpu.CompilerParams(use_tc_tiling_on_sc=False),
        scratch_types=[
            pltpu.VMEM((C,), i32),
            pltpu.VMEM((C, 8), f32),
            pltpu.VMEM_SHARED((NPAD, 8), f32),
        ],
    )


def _node_body(h_ref, a0_ref, a1_ref, t0_ref, t1_ref, coord_ref,
               wn1a_ref, wn1b_ref, bn1_ref, wn2_ref, bn2_ref,
               hout_ref, cout_ref):
    agg = a0_ref[0] + a1_ref[0]
    u = jnp.maximum(
        jnp.dot(h_ref[...], wn1a_ref[...], preferred_element_type=f32) +
        jnp.dot(agg, wn1b_ref[...], preferred_element_type=f32) +
        bn1_ref[...], 0.0)
    hout_ref[...] = (jnp.dot(u, wn2_ref[...], preferred_element_type=f32) +
                     bn2_ref[...])
    t = t0_ref[0] + t1_ref[0]
    s3 = t[:, 0:3]
    cnt = t[:, 3:4]
    cout_ref[...] = coord_ref[...] + s3 / jnp.maximum(cnt, 1.0)


def _node_model(h, agg_p, t_p, coord, wn1a, wn1b, bn1, wn2, bn2):
    bn = 1000
    wfull = lambda shape: pl.BlockSpec(shape, lambda i: (0, 0))
    return pl.pallas_call(
        _node_body,
        grid=(N // bn,),
        in_specs=[
            pl.BlockSpec((bn, 128), lambda i: (i, 0)),
            pl.BlockSpec((1, bn, 128), lambda i: (0, i, 0)),
            pl.BlockSpec((1, bn, 128), lambda i: (1, i, 0)),
            pl.BlockSpec((1, bn, 8), lambda i: (0, i, 0)),
            pl.BlockSpec((1, bn, 8), lambda i: (1, i, 0)),
            pl.BlockSpec((bn, 3), lambda i: (i, 0)),
            wfull((128, 128)), wfull((128, 128)), wfull((1, 128)),
            wfull((128, 128)), wfull((1, 128)),
        ],
        out_specs=[
            pl.BlockSpec((bn, 128), lambda i: (i, 0)),
            pl.BlockSpec((bn, 3), lambda i: (i, 0)),
        ],
        out_shape=[
            jax.ShapeDtypeStruct((N, 128), f32),
            jax.ShapeDtypeStruct((N, 3), f32),
        ],
    )(h, agg_p, agg_p, t_p, t_p, coord, wn1a, wn1b, bn1, wn2, bn2)


def kernel(h, edge_index, coord, edge_attr,
           W_e1, b_e1, W_e2, b_e2,
           W_n1, b_n1, W_n2, b_n2,
           W_c1, b_c1, W_c2, b_c2):
    row = edge_index[0]
    col = edge_index[1]
    c16 = jnp.pad(coord, ((0, 0), (0, 13)))
    w1a = W_e1[0:D]
    w1b = W_e1[D:2 * D]
    wr = W_e1[2 * D:2 * D + 1]
    wea = W_e1[2 * D + 1:]
    ta, tb = _prep_tables(h, w1a, w1b)
    pre = _gather_pre()(ta, tb, row, col)
    dr = _coord_feats()(c16, row, col)
    m_ij, t8 = _edge_mlp(pre, dr, edge_attr,
                         wr, wea, b_e1.reshape(1, H),
                         W_e2, b_e2.reshape(1, H),
                         W_c1, b_c1.reshape(1, H),
                         W_c2, b_c2.reshape(1, 1))
    z128 = jnp.zeros((NPAD, 128), f32)
    z16 = jnp.zeros((NPAD, 8), f32)
    agg_p = _scatter_m()(m_ij, row, z128)
    t_p = _scatter_t()(t8, row, z16)
    h_out, coord_out = _node_model(h, agg_p, t_p, coord,
                                   W_n1[0:D], W_n1[D:], b_n1.reshape(1, H),
                                   W_n2, b_n2.reshape(1, H))
    return (h_out, coord_out, m_ij)


# speedup vs baseline: 1.1267x; 1.1267x over previous
import functools

import jax
import jax.numpy as jnp
from jax import lax
from jax.experimental import pallas as pl
from jax.experimental.pallas import tpu as pltpu
from jax.experimental.pallas import tpu_sc as plsc

N, E, D, DE, H = 10000, 320000, 128, 4, 128
NPAD = 10240
NC, NS = 2, 16
NW = NC * NS
PER_W = E // NW
C = 80
CH = PER_W // C
ROWS_PER_TILE = NPAD // NS

f32 = jnp.float32
i32 = jnp.int32


def _prep_body(h_ref, w1a_ref, w1b_ref, ta_ref, tb_ref):
    hb = h_ref[...]
    ta_ref[...] = jnp.dot(hb, w1a_ref[...], preferred_element_type=f32)
    tb_ref[...] = jnp.dot(hb, w1b_ref[...], preferred_element_type=f32)


def _prep_tables(h, w1a, w1b):
    bn = 1000
    return pl.pallas_call(
        _prep_body,
        grid=(N // bn,),
        in_specs=[
            pl.BlockSpec((bn, 128), lambda i: (i, 0)),
            pl.BlockSpec((128, 128), lambda i: (0, 0)),
            pl.BlockSpec((128, 128), lambda i: (0, 0)),
        ],
        out_specs=[
            pl.BlockSpec((bn, 128), lambda i: (i, 0)),
            pl.BlockSpec((bn, 128), lambda i: (i, 0)),
        ],
        out_shape=[
            jax.ShapeDtypeStruct((N, 128), f32),
            jax.ShapeDtypeStruct((N, 128), f32),
        ],
    )(h, w1a, w1b)


def _gather_pre_body(ta, tb, row, col, out, idxr, idxc, bufr, bufc,
                     semr, semc):
    c = lax.axis_index("c")
    s = lax.axis_index("s")
    wid = s * NC + c
    base = wid * PER_W

    def chunk(k, carry):
        b = base + k * C
        pltpu.sync_copy(row.at[pl.ds(b, C)], idxr)
        pltpu.sync_copy(col.at[pl.ds(b, C)], idxc)
        cp1 = pltpu.async_copy(ta.at[idxr], bufr, semr)
        cp2 = pltpu.async_copy(tb.at[idxc], bufc, semc)
        cp1.wait()
        cp2.wait()

        def rowfn(i, carry2):
            for j in range(8):
                sl = pl.ds(j * 16, 16)
                bufr[i, sl] = bufr[i, sl] + bufc[i, sl]
            return carry2

        lax.fori_loop(0, C, rowfn, 0)
        pltpu.sync_copy(bufr, out.at[pl.ds(b, C)])
        return carry

    lax.fori_loop(0, CH, chunk, 0)


@functools.cache
def _gather_pre():
    return pl.kernel(
        _gather_pre_body,
        out_type=jax.ShapeDtypeStruct((E, 128), f32),
        mesh=plsc.VectorSubcoreMesh(core_axis_name="c", subcore_axis_name="s",
                                    num_cores=NC, num_subcores=NS),
        scratch_types=[
            pltpu.VMEM((C,), i32),
            pltpu.VMEM((C,), i32),
            pltpu.VMEM((C, 128), f32),
            pltpu.VMEM((C, 128), f32),
            pltpu.SemaphoreType.DMA,
            pltpu.SemaphoreType.DMA,
        ],
    )


def _coord_feats_body(c16, row, col, out, idxr, idxc, bufr, bufc,
                      semr, semc):
    c = lax.axis_index("c")
    s = lax.axis_index("s")
    wid = s * NC + c
    base = wid * PER_W
    pltpu.sync_copy(row.at[pl.ds(base, PER_W)], idxr)
    pltpu.sync_copy(col.at[pl.ds(base, PER_W)], idxc)

    def chunk(k, carry):
        b = base + k * C
        cp1 = pltpu.async_copy(c16.at[idxr.at[pl.ds(k * C, C)]], bufr, semr)
        cp2 = pltpu.async_copy(c16.at[idxc.at[pl.ds(k * C, C)]], bufc, semc)
        cp1.wait()
        cp2.wait()

        def rowfn(i, carry2):
            bufr[i, :] = bufr[i, :] - bufc[i, :]
            return carry2

        lax.fori_loop(0, C, rowfn, 0)
        pltpu.sync_copy(bufr, out.at[pl.ds(b, C)])
        return carry

    lax.fori_loop(0, CH, chunk, 0)


@functools.cache
def _coord_feats():
    return pl.kernel(
        _coord_feats_body,
        out_type=jax.ShapeDtypeStruct((E, 16), f32),
        mesh=plsc.VectorSubcoreMesh(core_axis_name="c", subcore_axis_name="s",
                                    num_cores=NC, num_subcores=NS),
        compiler_params=pltpu.CompilerParams(use_tc_tiling_on_sc=False,
                                             needs_layout_passes=False),
        scratch_types=[
            pltpu.VMEM((PER_W,), i32),
            pltpu.VMEM((PER_W,), i32),
            pltpu.VMEM((C, 16), f32),
            pltpu.VMEM((C, 16), f32),
            pltpu.SemaphoreType.DMA,
            pltpu.SemaphoreType.DMA,
        ],
    )


def _edge_body(pre_ref, dr_ref, ea_ref, wr_ref, wea_ref, be1_ref,
               we2_ref, be2_ref, wc1_ref, bc1_ref, wc2_ref, bc2_ref,
               m_ref, t8_ref):
    dr = dr_ref[...]
    rad = jnp.sum(dr * dr, axis=1, keepdims=True)
    x1 = (pre_ref[...] + rad * wr_ref[...] +
          jnp.dot(ea_ref[...], wea_ref[...], preferred_element_type=f32) +
          be1_ref[...])
    x1 = jnp.maximum(x1, 0.0)
    m = jnp.maximum(
        jnp.dot(x1, we2_ref[...], preferred_element_type=f32) + be2_ref[...],
        0.0)
    m_ref[...] = m
    cfh = jnp.maximum(
        jnp.dot(m, wc1_ref[...], preferred_element_type=f32) + bc1_ref[...],
        0.0)
    cf = jnp.dot(cfh, wc2_ref[...], preferred_element_type=f32) + bc2_ref[...]
    t = dr[:, 0:8] * cf
    iot = lax.broadcasted_iota(i32, t.shape, 1)
    t8_ref[...] = jnp.where(iot == 3, 1.0, t)


def _edge_mlp(pre, dr, edge_attr, wr, wea, be1, we2, be2, wc1, bc1, wc2, bc2):
    be = 2000
    wfull = lambda shape: pl.BlockSpec(shape, lambda i: (0, 0))
    return pl.pallas_call(
        _edge_body,
        grid=(E // be,),
        in_specs=[
            pl.BlockSpec((be, 128), lambda i: (i, 0)),
            pl.BlockSpec((be, 16), lambda i: (i, 0)),
            pl.BlockSpec((be, DE), lambda i: (i, 0)),
            wfull((1, 128)), wfull((DE, 128)), wfull((1, 128)),
            wfull((128, 128)), wfull((1, 128)),
            wfull((128, 128)), wfull((1, 128)),
            wfull((128, 1)), wfull((1, 1)),
        ],
        out_specs=[
            pl.BlockSpec((be, 128), lambda i: (i, 0)),
            pl.BlockSpec((be, 8), lambda i: (i, 0)),
        ],
        out_shape=[
            jax.ShapeDtypeStruct((E, 128), f32),
            jax.ShapeDtypeStruct((E, 8), f32),
        ],
    )(pre, dr, edge_attr, wr, wea, be1, we2, be2, wc1, bc1, wc2, bc2)


def _scatter_m_body(mij, row, z128, agg_out, idxv, mbuf, aggsh):
    c = lax.axis_index("c")
    s = lax.axis_index("s")
    wid = s * NC + c
    base = wid * PER_W
    rsl = pl.ds(s * ROWS_PER_TILE, ROWS_PER_TILE)
    pltpu.sync_copy(z128.at[rsl], aggsh.at[rsl])
    plsc.subcore_barrier()

    def chunk(k, carry):
        b = base + k * C
        pltpu.sync_copy(row.at[pl.ds(b, C)], idxv)
        pltpu.sync_copy(mij.at[pl.ds(b, C)], mbuf)
        pltpu.sync_copy(mbuf, aggsh.at[idxv], add=True)
        return carry

    lax.fori_loop(0, CH, chunk, 0)
    plsc.subcore_barrier()
    pltpu.sync_copy(aggsh.at[rsl], agg_out.at[c].at[rsl])


@functools.cache
def _scatter_m():
    return pl.kernel(
        _scatter_m_body,
        out_type=jax.ShapeDtypeStruct((NC, NPAD, 128), f32),
        mesh=plsc.VectorSubcoreMesh(core_axis_name="c", subcore_axis_name="s",
                                    num_cores=NC, num_subcores=NS),
        scratch_types=[
            pltpu.VMEM((C,), i32),
            pltpu.VMEM((C, 128), f32),
            pltpu.VMEM_SHARED((NPAD, 128), f32),
        ],
    )


def _scatter_t_body(t8, row, z16, t_out, idxv, tbuf, tsh):
    c = lax.axis_index("c")
    s = lax.axis_index("s")
    wid = s * NC + c
    base = wid * PER_W
    rsl = pl.ds(s * ROWS_PER_TILE, ROWS_PER_TILE)
    pltpu.sync_copy(z16.at[rsl], tsh.at[rsl])
    plsc.subcore_barrier()

    def chunk(k, carry):
        b = base + k * C
        pltpu.sync_copy(row.at[pl.ds(b, C)], idxv)
        pltpu.sync_copy(t8.at[pl.ds(b, C)], tbuf)
        pltpu.sync_copy(tbuf, tsh.at[idxv], add=True)
        return carry

    lax.fori_loop(0, CH, chunk, 0)
    plsc.subcore_barrier()
    pltpu.sync_copy(tsh.at[rsl], t_out.at[c].at[rsl])


@functools.cache
def _scatter_t():
    return pl.kernel(
        _scatter_t_body,
        out_type=jax.ShapeDtypeStruct((NC, NPAD, 8), f32),
        mesh=plsc.VectorSubcoreMesh(core_axis_name="c", subcore_axis_name="s",
                                    num_cores=NC, num_subcores=NS),
        compiler_params=pltpu.CompilerParams(use_tc_tiling_on_sc=False),
        scratch_types=[
            pltpu.VMEM((C,), i32),
            pltpu.VMEM((C, 8), f32),
            pltpu.VMEM_SHARED((NPAD, 8), f32),
        ],
    )


def _node_body(h_ref, a0_ref, a1_ref, t0_ref, t1_ref, coord_ref,
               wn1a_ref, wn1b_ref, bn1_ref, wn2_ref, bn2_ref,
               hout_ref, cout_ref):
    agg = a0_ref[0] + a1_ref[0]
    u = jnp.maximum(
        jnp.dot(h_ref[...], wn1a_ref[...], preferred_element_type=f32) +
        jnp.dot(agg, wn1b_ref[...], preferred_element_type=f32) +
        bn1_ref[...], 0.0)
    hout_ref[...] = (jnp.dot(u, wn2_ref[...], preferred_element_type=f32) +
                     bn2_ref[...])
    t = t0_ref[0] + t1_ref[0]
    s3 = t[:, 0:3]
    cnt = t[:, 3:4]
    cout_ref[...] = coord_ref[...] + s3 / jnp.maximum(cnt, 1.0)


def _node_model(h, agg_p, t_p, coord, wn1a, wn1b, bn1, wn2, bn2):
    bn = 1000
    wfull = lambda shape: pl.BlockSpec(shape, lambda i: (0, 0))
    return pl.pallas_call(
        _node_body,
        grid=(N // bn,),
        in_specs=[
            pl.BlockSpec((bn, 128), lambda i: (i, 0)),
            pl.BlockSpec((1, bn, 128), lambda i: (0, i, 0)),
            pl.BlockSpec((1, bn, 128), lambda i: (1, i, 0)),
            pl.BlockSpec((1, bn, 8), lambda i: (0, i, 0)),
            pl.BlockSpec((1, bn, 8), lambda i: (1, i, 0)),
            pl.BlockSpec((bn, 3), lambda i: (i, 0)),
            wfull((128, 128)), wfull((128, 128)), wfull((1, 128)),
            wfull((128, 128)), wfull((1, 128)),
        ],
        out_specs=[
            pl.BlockSpec((bn, 128), lambda i: (i, 0)),
            pl.BlockSpec((bn, 3), lambda i: (i, 0)),
        ],
        out_shape=[
            jax.ShapeDtypeStruct((N, 128), f32),
            jax.ShapeDtypeStruct((N, 3), f32),
        ],
    )(h, agg_p, agg_p, t_p, t_p, coord, wn1a, wn1b, bn1, wn2, bn2)


def kernel(h, edge_index, coord, edge_attr,
           W_e1, b_e1, W_e2, b_e2,
           W_n1, b_n1, W_n2, b_n2,
           W_c1, b_c1, W_c2, b_c2):
    row = edge_index[0]
    col = edge_index[1]
    c16 = jnp.pad(coord, ((0, 0), (0, 13)))
    w1a = W_e1[0:D]
    w1b = W_e1[D:2 * D]
    wr = W_e1[2 * D:2 * D + 1]
    wea = W_e1[2 * D + 1:]
    ta, tb = _prep_tables(h, w1a, w1b)
    pre = _gather_pre()(ta, tb, row, col)
    dr = _coord_feats()(c16, row, col)
    m_ij, t8 = _edge_mlp(pre, dr, edge_attr,
                         wr, wea, b_e1.reshape(1, H),
                         W_e2, b_e2.reshape(1, H),
                         W_c1, b_c1.reshape(1, H),
                         W_c2, b_c2.reshape(1, 1))
    z128 = jnp.zeros((NPAD, 128), f32)
    z16 = jnp.zeros((NPAD, 8), f32)
    agg_p = _scatter_m()(m_ij, row, z128)
    t_p = _scatter_t()(t8, row, z16)
    h_out, coord_out = _node_model(h, agg_p, t_p, coord,
                                   W_n1[0:D], W_n1[D:], b_n1.reshape(1, H),
                                   W_n2, b_n2.reshape(1, H))
    return (h_out, coord_out, m_ij)


# speedup vs baseline: 1.6346x; 1.4508x over previous
import functools

import jax
import jax.numpy as jnp
from jax import lax
from jax.experimental import pallas as pl
from jax.experimental.pallas import tpu as pltpu
from jax.experimental.pallas import tpu_sc as plsc

N, E, D, DE, H = 10000, 320000, 128, 4, 128
NPAD = 10240
NC, NS = 2, 16
NW = NC * NS
PER_W = E // NW
C = 80
CH = PER_W // C
ROWS_PER_TILE = NPAD // NS

f32 = jnp.float32
i32 = jnp.int32


def _prep_body(h_ref, w1a_ref, w1b_ref, ta_ref, tb_ref):
    hb = h_ref[...]
    ta_ref[...] = jnp.dot(hb, w1a_ref[...], preferred_element_type=f32)
    tb_ref[...] = jnp.dot(hb, w1b_ref[...], preferred_element_type=f32)


def _prep_tables(h, w1a, w1b):
    bn = 1000
    return pl.pallas_call(
        _prep_body,
        grid=(N // bn,),
        in_specs=[
            pl.BlockSpec((bn, 128), lambda i: (i, 0)),
            pl.BlockSpec((128, 128), lambda i: (0, 0)),
            pl.BlockSpec((128, 128), lambda i: (0, 0)),
        ],
        out_specs=[
            pl.BlockSpec((bn, 128), lambda i: (i, 0)),
            pl.BlockSpec((bn, 128), lambda i: (i, 0)),
        ],
        out_shape=[
            jax.ShapeDtypeStruct((N, 128), f32),
            jax.ShapeDtypeStruct((N, 128), f32),
        ],
    )(h, w1a, w1b)


def _gather_pre_body(ta, tb, row, col, out, idxr, idxc, bufr, bufc,
                     semr0, semr1, semc0, semc1):
    c = lax.axis_index("c")
    s = lax.axis_index("s")
    wid = s * NC + c
    base = wid * PER_W
    pltpu.sync_copy(row.at[pl.ds(base, PER_W)], idxr)
    pltpu.sync_copy(col.at[pl.ds(base, PER_W)], idxc)
    semr = (semr0, semr1)
    semc = (semc0, semc1)

    def start(k, p):
        pltpu.async_copy(ta.at[idxr.at[pl.ds(k * C, C)]], bufr.at[p], semr[p])
        pltpu.async_copy(tb.at[idxc.at[pl.ds(k * C, C)]], bufc.at[p], semc[p])

    def wait(p):
        pltpu.make_async_copy(ta.at[idxr.at[pl.ds(0, C)]], bufr.at[p],
                              semr[p]).wait()
        pltpu.make_async_copy(tb.at[idxc.at[pl.ds(0, C)]], bufc.at[p],
                              semc[p]).wait()

    start(0, 0)
    start(1, 1)

    def chunk2(k2, carry):
        for p in range(2):
            k = k2 * 2 + p
            wait(p)

            def rowfn(i, carry2):
                for j in range(8):
                    sl = pl.ds(j * 16, 16)
                    bufr[p, i, sl] = bufr[p, i, sl] + bufc[p, i, sl]
                return carry2

            lax.fori_loop(0, C, rowfn, 0)
            pltpu.sync_copy(bufr.at[p], out.at[pl.ds(base + k * C, C)])

            @pl.when(k + 2 < CH)
            def _():
                start(k + 2, p)
        return carry

    lax.fori_loop(0, CH // 2, chunk2, 0)

    @pl.when((CH % 2) == 1)
    def _():
        wait(0)

        def rowfn(i, carry2):
            for j in range(8):
                sl = pl.ds(j * 16, 16)
                bufr[0, i, sl] = bufr[0, i, sl] + bufc[0, i, sl]
            return carry2

        lax.fori_loop(0, C, rowfn, 0)
        pltpu.sync_copy(bufr.at[0], out.at[pl.ds(base + (CH - 1) * C, C)])


@functools.cache
def _gather_pre():
    return pl.kernel(
        _gather_pre_body,
        out_type=jax.ShapeDtypeStruct((E, 128), f32),
        mesh=plsc.VectorSubcoreMesh(core_axis_name="c", subcore_axis_name="s",
                                    num_cores=NC, num_subcores=NS),
        scratch_types=[
            pltpu.VMEM((PER_W,), i32),
            pltpu.VMEM((PER_W,), i32),
            pltpu.VMEM((2, C, 128), f32),
            pltpu.VMEM((2, C, 128), f32),
            pltpu.SemaphoreType.DMA,
            pltpu.SemaphoreType.DMA,
            pltpu.SemaphoreType.DMA,
            pltpu.SemaphoreType.DMA,
        ],
    )


def _coord_feats_body(c16, row, col, out, idxr, idxc, bufr, bufc,
                      semr0, semr1, semc0, semc1):
    c = lax.axis_index("c")
    s = lax.axis_index("s")
    wid = s * NC + c
    base = wid * PER_W
    pltpu.sync_copy(row.at[pl.ds(base, PER_W)], idxr)
    pltpu.sync_copy(col.at[pl.ds(base, PER_W)], idxc)
    semrs = (semr0, semr1)
    semcs = (semc0, semc1)

    def start(k, p):
        pltpu.async_copy(c16.at[idxr.at[pl.ds(k * C, C)]], bufr.at[p],
                         semrs[p])
        pltpu.async_copy(c16.at[idxc.at[pl.ds(k * C, C)]], bufc.at[p],
                         semcs[p])

    def wait(p):
        pltpu.make_async_copy(c16.at[idxr.at[pl.ds(0, C)]], bufr.at[p],
                              semrs[p]).wait()
        pltpu.make_async_copy(c16.at[idxc.at[pl.ds(0, C)]], bufc.at[p],
                              semcs[p]).wait()

    def body(k, p):
        wait(p)

        def rowfn(i, carry2):
            bufr[p, i, :] = bufr[p, i, :] - bufc[p, i, :]
            return carry2

        lax.fori_loop(0, C, rowfn, 0)
        pltpu.sync_copy(bufr.at[p], out.at[pl.ds(base + k * C, C)])

    start(0, 0)
    start(1, 1)

    def chunk2(k2, carry):
        for p in range(2):
            k = k2 * 2 + p
            body(k, p)

            @pl.when(k + 2 < CH)
            def _():
                start(k + 2, p)
        return carry

    lax.fori_loop(0, CH // 2, chunk2, 0)

    @pl.when((CH % 2) == 1)
    def _():
        body(CH - 1, 0)


@functools.cache
def _coord_feats():
    return pl.kernel(
        _coord_feats_body,
        out_type=jax.ShapeDtypeStruct((E, 16), f32),
        mesh=plsc.VectorSubcoreMesh(core_axis_name="c", subcore_axis_name="s",
                                    num_cores=NC, num_subcores=NS),
        compiler_params=pltpu.CompilerParams(use_tc_tiling_on_sc=False,
                                             needs_layout_passes=False),
        scratch_types=[
            pltpu.VMEM((PER_W,), i32),
            pltpu.VMEM((PER_W,), i32),
            pltpu.VMEM((2, C, 16), f32),
            pltpu.VMEM((2, C, 16), f32),
            pltpu.SemaphoreType.DMA,
            pltpu.SemaphoreType.DMA,
            pltpu.SemaphoreType.DMA,
            pltpu.SemaphoreType.DMA,
        ],
    )


def _edge_body(pre_ref, dr_ref, ea_ref, wr_ref, wea_ref, be1_ref,
               we2_ref, be2_ref, wc1_ref, bc1_ref, wc2_ref, bc2_ref,
               m_ref, t8_ref):
    dr = dr_ref[...]
    rad = jnp.sum(dr * dr, axis=1, keepdims=True)
    x1 = (pre_ref[...] + rad * wr_ref[...] +
          jnp.dot(ea_ref[...], wea_ref[...], preferred_element_type=f32) +
          be1_ref[...])
    x1 = jnp.maximum(x1, 0.0)
    m = jnp.maximum(
        jnp.dot(x1, we2_ref[...], preferred_element_type=f32) + be2_ref[...],
        0.0)
    m_ref[...] = m
    cfh = jnp.maximum(
        jnp.dot(m, wc1_ref[...], preferred_element_type=f32) + bc1_ref[...],
        0.0)
    cf = jnp.dot(cfh, wc2_ref[...], preferred_element_type=f32) + bc2_ref[...]
    t = dr[:, 0:8] * cf
    iot = lax.broadcasted_iota(i32, t.shape, 1)
    t8_ref[...] = jnp.where(iot == 3, 1.0, t)


def _edge_mlp(pre, dr, edge_attr, wr, wea, be1, we2, be2, wc1, bc1, wc2, bc2):
    be = 2000
    wfull = lambda shape: pl.BlockSpec(shape, lambda i: (0, 0))
    return pl.pallas_call(
        _edge_body,
        grid=(E // be,),
        in_specs=[
            pl.BlockSpec((be, 128), lambda i: (i, 0)),
            pl.BlockSpec((be, 16), lambda i: (i, 0)),
            pl.BlockSpec((be, DE), lambda i: (i, 0)),
            wfull((1, 128)), wfull((DE, 128)), wfull((1, 128)),
            wfull((128, 128)), wfull((1, 128)),
            wfull((128, 128)), wfull((1, 128)),
            wfull((128, 1)), wfull((1, 1)),
        ],
        out_specs=[
            pl.BlockSpec((be, 128), lambda i: (i, 0)),
            pl.BlockSpec((be, 8), lambda i: (i, 0)),
        ],
        out_shape=[
            jax.ShapeDtypeStruct((E, 128), f32),
            jax.ShapeDtypeStruct((E, 8), f32),
        ],
    )(pre, dr, edge_attr, wr, wea, be1, we2, be2, wc1, bc1, wc2, bc2)


def _scatter_m_body(mij, row, z128, agg_out, idxv, mbuf, aggsh, semm0, semm1):
    c = lax.axis_index("c")
    s = lax.axis_index("s")
    wid = s * NC + c
    base = wid * PER_W
    rsl = pl.ds(s * ROWS_PER_TILE, ROWS_PER_TILE)
    pltpu.sync_copy(z128.at[rsl], aggsh.at[rsl])
    plsc.subcore_barrier()
    sems = (semm0, semm1)

    def start(k, p):
        b = base + k * C
        pltpu.sync_copy(row.at[pl.ds(b, C)], idxv.at[p])
        pltpu.async_copy(mij.at[pl.ds(b, C)], mbuf.at[p], sems[p])

    def body(p):
        pltpu.make_async_copy(mij.at[pl.ds(0, C)], mbuf.at[p],
                              sems[p]).wait()
        pltpu.sync_copy(mbuf.at[p], aggsh.at[idxv.at[p]], add=True)

    start(0, 0)
    start(1, 1)

    def chunk2(k2, carry):
        for p in range(2):
            k = k2 * 2 + p
            body(p)

            @pl.when(k + 2 < CH)
            def _():
                start(k + 2, p)
        return carry

    lax.fori_loop(0, CH // 2, chunk2, 0)

    @pl.when((CH % 2) == 1)
    def _():
        body(0)

    plsc.subcore_barrier()
    pltpu.sync_copy(aggsh.at[rsl], agg_out.at[c].at[rsl])


@functools.cache
def _scatter_m():
    return pl.kernel(
        _scatter_m_body,
        out_type=jax.ShapeDtypeStruct((NC, NPAD, 128), f32),
        mesh=plsc.VectorSubcoreMesh(core_axis_name="c", subcore_axis_name="s",
                                    num_cores=NC, num_subcores=NS),
        scratch_types=[
            pltpu.VMEM((2, C), i32),
            pltpu.VMEM((2, C, 128), f32),
            pltpu.VMEM_SHARED((NPAD, 128), f32),
            pltpu.SemaphoreType.DMA,
            pltpu.SemaphoreType.DMA,
        ],
    )


def _scatter_t_body(t8, row, z16, t_out, idxv, tbuf, tsh, semt0, semt1):
    c = lax.axis_index("c")
    s = lax.axis_index("s")
    wid = s * NC + c
    base = wid * PER_W
    rsl = pl.ds(s * ROWS_PER_TILE, ROWS_PER_TILE)
    pltpu.sync_copy(z16.at[rsl], tsh.at[rsl])
    plsc.subcore_barrier()
    sems = (semt0, semt1)

    def start(k, p):
        b = base + k * C
        pltpu.sync_copy(row.at[pl.ds(b, C)], idxv.at[p])
        pltpu.async_copy(t8.at[pl.ds(b, C)], tbuf.at[p], sems[p])

    def body(p):
        pltpu.make_async_copy(t8.at[pl.ds(0, C)], tbuf.at[p],
                              sems[p]).wait()
        pltpu.sync_copy(tbuf.at[p], tsh.at[idxv.at[p]], add=True)

    start(0, 0)
    start(1, 1)

    def chunk2(k2, carry):
        for p in range(2):
            k = k2 * 2 + p
            body(p)

            @pl.when(k + 2 < CH)
            def _():
                start(k + 2, p)
        return carry

    lax.fori_loop(0, CH // 2, chunk2, 0)

    @pl.when((CH % 2) == 1)
    def _():
        body(0)

    plsc.subcore_barrier()
    pltpu.sync_copy(tsh.at[rsl], t_out.at[c].at[rsl])


@functools.cache
def _scatter_t():
    return pl.kernel(
        _scatter_t_body,
        out_type=jax.ShapeDtypeStruct((NC, NPAD, 8), f32),
        mesh=plsc.VectorSubcoreMesh(core_axis_name="c", subcore_axis_name="s",
                                    num_cores=NC, num_subcores=NS),
        compiler_params=pltpu.CompilerParams(use_tc_tiling_on_sc=False),
        scratch_types=[
            pltpu.VMEM((2, C), i32),
            pltpu.VMEM((2, C, 8), f32),
            pltpu.VMEM_SHARED((NPAD, 8), f32),
            pltpu.SemaphoreType.DMA,
            pltpu.SemaphoreType.DMA,
        ],
    )


def _node_body(h_ref, a0_ref, a1_ref, t0_ref, t1_ref, coord_ref,
               wn1a_ref, wn1b_ref, bn1_ref, wn2_ref, bn2_ref,
               hout_ref, cout_ref):
    agg = a0_ref[0] + a1_ref[0]
    u = jnp.maximum(
        jnp.dot(h_ref[...], wn1a_ref[...], preferred_element_type=f32) +
        jnp.dot(agg, wn1b_ref[...], preferred_element_type=f32) +
        bn1_ref[...], 0.0)
    hout_ref[...] = (jnp.dot(u, wn2_ref[...], preferred_element_type=f32) +
                     bn2_ref[...])
    t = t0_ref[0] + t1_ref[0]
    s3 = t[:, 0:3]
    cnt = t[:, 3:4]
    cout_ref[...] = coord_ref[...] + s3 / jnp.maximum(cnt, 1.0)


def _node_model(h, agg_p, t_p, coord, wn1a, wn1b, bn1, wn2, bn2):
    bn = 1000
    wfull = lambda shape: pl.BlockSpec(shape, lambda i: (0, 0))
    return pl.pallas_call(
        _node_body,
        grid=(N // bn,),
        in_specs=[
            pl.BlockSpec((bn, 128), lambda i: (i, 0)),
            pl.BlockSpec((1, bn, 128), lambda i: (0, i, 0)),
            pl.BlockSpec((1, bn, 128), lambda i: (1, i, 0)),
            pl.BlockSpec((1, bn, 8), lambda i: (0, i, 0)),
            pl.BlockSpec((1, bn, 8), lambda i: (1, i, 0)),
            pl.BlockSpec((bn, 3), lambda i: (i, 0)),
            wfull((128, 128)), wfull((128, 128)), wfull((1, 128)),
            wfull((128, 128)), wfull((1, 128)),
        ],
        out_specs=[
            pl.BlockSpec((bn, 128), lambda i: (i, 0)),
            pl.BlockSpec((bn, 3), lambda i: (i, 0)),
        ],
        out_shape=[
            jax.ShapeDtypeStruct((N, 128), f32),
            jax.ShapeDtypeStruct((N, 3), f32),
        ],
    )(h, agg_p, agg_p, t_p, t_p, coord, wn1a, wn1b, bn1, wn2, bn2)


def kernel(h, edge_index, coord, edge_attr,
           W_e1, b_e1, W_e2, b_e2,
           W_n1, b_n1, W_n2, b_n2,
           W_c1, b_c1, W_c2, b_c2):
    row = edge_index[0]
    col = edge_index[1]
    c16 = jnp.pad(coord, ((0, 0), (0, 13)))
    w1a = W_e1[0:D]
    w1b = W_e1[D:2 * D]
    wr = W_e1[2 * D:2 * D + 1]
    wea = W_e1[2 * D + 1:]
    ta, tb = _prep_tables(h, w1a, w1b)
    pre = _gather_pre()(ta, tb, row, col)
    dr = _coord_feats()(c16, row, col)
    m_ij, t8 = _edge_mlp(pre, dr, edge_attr,
                         wr, wea, b_e1.reshape(1, H),
                         W_e2, b_e2.reshape(1, H),
                         W_c1, b_c1.reshape(1, H),
                         W_c2, b_c2.reshape(1, 1))
    z128 = jnp.zeros((NPAD, 128), f32)
    z16 = jnp.zeros((NPAD, 8), f32)
    agg_p = _scatter_m()(m_ij, row, z128)
    t_p = _scatter_t()(t8, row, z16)
    h_out, coord_out = _node_model(h, agg_p, t_p, coord,
                                   W_n1[0:D], W_n1[D:], b_n1.reshape(1, H),
                                   W_n2, b_n2.reshape(1, H))
    return (h_out, coord_out, m_ij)


# speedup vs baseline: 1.6572x; 1.0138x over previous
import functools

import jax
import jax.numpy as jnp
from jax import lax
from jax.experimental import pallas as pl
from jax.experimental.pallas import tpu as pltpu
from jax.experimental.pallas import tpu_sc as plsc

N, E, D, DE, H = 10000, 320000, 128, 4, 128
NPAD = 10240
NC, NS = 2, 16
NW = NC * NS
PER_W = E // NW
C = 80
CH = PER_W // C
ROWS_PER_TILE = NPAD // NS

f32 = jnp.float32
i32 = jnp.int32


def _prep_body(h_ref, w1a_ref, w1b_ref, ta_ref, tb_ref):
    hb = h_ref[...]
    ta_ref[...] = jnp.dot(hb, w1a_ref[...], preferred_element_type=f32)
    tb_ref[...] = jnp.dot(hb, w1b_ref[...], preferred_element_type=f32)


def _prep_tables(h, w1a, w1b):
    bn = 1000
    return pl.pallas_call(
        _prep_body,
        grid=(N // bn,),
        in_specs=[
            pl.BlockSpec((bn, 128), lambda i: (i, 0)),
            pl.BlockSpec((128, 128), lambda i: (0, 0)),
            pl.BlockSpec((128, 128), lambda i: (0, 0)),
        ],
        out_specs=[
            pl.BlockSpec((bn, 128), lambda i: (i, 0)),
            pl.BlockSpec((bn, 128), lambda i: (i, 0)),
        ],
        out_shape=[
            jax.ShapeDtypeStruct((N, 128), f32),
            jax.ShapeDtypeStruct((N, 128), f32),
        ],
    )(h, w1a, w1b)


def _gather_pre_body(ta, tb, row, col, out, idxr, idxc, bufr, bufc,
                     semr0, semr1, semc0, semc1):
    c = lax.axis_index("c")
    s = lax.axis_index("s")
    wid = s * NC + c
    base = wid * PER_W
    pltpu.sync_copy(row.at[pl.ds(base, PER_W)], idxr)
    pltpu.sync_copy(col.at[pl.ds(base, PER_W)], idxc)
    semr = (semr0, semr1)
    semc = (semc0, semc1)

    def start(k, p):
        pltpu.async_copy(ta.at[idxr.at[pl.ds(k * C, C)]], bufr.at[p], semr[p])
        pltpu.async_copy(tb.at[idxc.at[pl.ds(k * C, C)]], bufc.at[p], semc[p])

    def wait(p):
        pltpu.make_async_copy(ta.at[idxr.at[pl.ds(0, C)]], bufr.at[p],
                              semr[p]).wait()
        pltpu.make_async_copy(tb.at[idxc.at[pl.ds(0, C)]], bufc.at[p],
                              semc[p]).wait()

    start(0, 0)
    start(1, 1)

    def chunk2(k2, carry):
        for p in range(2):
            k = k2 * 2 + p
            wait(p)

            def rowfn(i, carry2):
                for j in range(8):
                    sl = pl.ds(j * 16, 16)
                    bufr[p, i, sl] = bufr[p, i, sl] + bufc[p, i, sl]
                return carry2

            lax.fori_loop(0, C, rowfn, 0)
            pltpu.sync_copy(bufr.at[p], out.at[pl.ds(base + k * C, C)])

            @pl.when(k + 2 < CH)
            def _():
                start(k + 2, p)
        return carry

    lax.fori_loop(0, CH // 2, chunk2, 0)

    @pl.when((CH % 2) == 1)
    def _():
        wait(0)

        def rowfn(i, carry2):
            for j in range(8):
                sl = pl.ds(j * 16, 16)
                bufr[0, i, sl] = bufr[0, i, sl] + bufc[0, i, sl]
            return carry2

        lax.fori_loop(0, C, rowfn, 0)
        pltpu.sync_copy(bufr.at[0], out.at[pl.ds(base + (CH - 1) * C, C)])


@functools.cache
def _gather_pre():
    return pl.kernel(
        _gather_pre_body,
        out_type=jax.ShapeDtypeStruct((E, 128), f32),
        mesh=plsc.VectorSubcoreMesh(core_axis_name="c", subcore_axis_name="s",
                                    num_cores=NC, num_subcores=NS),
        scratch_types=[
            pltpu.VMEM((PER_W,), i32),
            pltpu.VMEM((PER_W,), i32),
            pltpu.VMEM((2, C, 128), f32),
            pltpu.VMEM((2, C, 128), f32),
            pltpu.SemaphoreType.DMA,
            pltpu.SemaphoreType.DMA,
            pltpu.SemaphoreType.DMA,
            pltpu.SemaphoreType.DMA,
        ],
    )


def _coord_feats_body(c16, row, col, out, idxr, idxc, bufr, bufc,
                      semr0, semr1, semc0, semc1):
    c = lax.axis_index("c")
    s = lax.axis_index("s")
    wid = s * NC + c
    base = wid * PER_W
    pltpu.sync_copy(row.at[pl.ds(base, PER_W)], idxr)
    pltpu.sync_copy(col.at[pl.ds(base, PER_W)], idxc)
    semrs = (semr0, semr1)
    semcs = (semc0, semc1)

    def start(k, p):
        pltpu.async_copy(c16.at[idxr.at[pl.ds(k * C, C)]], bufr.at[p],
                         semrs[p])
        pltpu.async_copy(c16.at[idxc.at[pl.ds(k * C, C)]], bufc.at[p],
                         semcs[p])

    def wait(p):
        pltpu.make_async_copy(c16.at[idxr.at[pl.ds(0, C)]], bufr.at[p],
                              semrs[p]).wait()
        pltpu.make_async_copy(c16.at[idxc.at[pl.ds(0, C)]], bufc.at[p],
                              semcs[p]).wait()

    def body(k, p):
        wait(p)

        def rowfn(i, carry2):
            bufr[p, i, :] = bufr[p, i, :] - bufc[p, i, :]
            return carry2

        lax.fori_loop(0, C, rowfn, 0)
        pltpu.sync_copy(bufr.at[p], out.at[pl.ds(base + k * C, C)])

    start(0, 0)
    start(1, 1)

    def chunk2(k2, carry):
        for p in range(2):
            k = k2 * 2 + p
            body(k, p)

            @pl.when(k + 2 < CH)
            def _():
                start(k + 2, p)
        return carry

    lax.fori_loop(0, CH // 2, chunk2, 0)

    @pl.when((CH % 2) == 1)
    def _():
        body(CH - 1, 0)


@functools.cache
def _coord_feats():
    return pl.kernel(
        _coord_feats_body,
        out_type=jax.ShapeDtypeStruct((E, 16), f32),
        mesh=plsc.VectorSubcoreMesh(core_axis_name="c", subcore_axis_name="s",
                                    num_cores=NC, num_subcores=NS),
        compiler_params=pltpu.CompilerParams(use_tc_tiling_on_sc=False,
                                             needs_layout_passes=False),
        scratch_types=[
            pltpu.VMEM((PER_W,), i32),
            pltpu.VMEM((PER_W,), i32),
            pltpu.VMEM((2, C, 16), f32),
            pltpu.VMEM((2, C, 16), f32),
            pltpu.SemaphoreType.DMA,
            pltpu.SemaphoreType.DMA,
            pltpu.SemaphoreType.DMA,
            pltpu.SemaphoreType.DMA,
        ],
    )


def _edge_body(pre_ref, dr_ref, ea_ref, wr_ref, wea_ref, be1_ref,
               we2_ref, be2_ref, wc1_ref, bc1_ref, wc2_ref, bc2_ref,
               m_ref, t8_ref):
    be = m_ref.shape[0]
    dr = dr_ref[...]
    rad = jnp.sum(dr * dr, axis=1, keepdims=True)
    x1 = (pre_ref[...] + rad * wr_ref[...] +
          jnp.dot(ea_ref[...], wea_ref[...], preferred_element_type=f32) +
          be1_ref[...])
    x1 = jnp.maximum(x1, 0.0)
    m = jnp.maximum(
        jnp.dot(x1, we2_ref[...], preferred_element_type=f32) + be2_ref[...],
        0.0)
    m_ref[...] = m
    cfh = jnp.maximum(
        jnp.dot(m, wc1_ref[...], preferred_element_type=f32) + bc1_ref[...],
        0.0)
    cf = jnp.dot(cfh, wc2_ref[...], preferred_element_type=f32) + bc2_ref[...]
    t = dr * cf
    iot = lax.broadcasted_iota(i32, t.shape, 1)
    t16 = jnp.where(iot == 3, 1.0, t)
    t8_ref[...] = jnp.concatenate([t16, jnp.zeros((be, 112), f32)], axis=1)


def _edge_mlp(pre, dr, edge_attr, wr, wea, be1, we2, be2, wc1, bc1, wc2, bc2):
    be = 1600
    wfull = lambda shape: pl.BlockSpec(shape, lambda i: (0, 0))
    return pl.pallas_call(
        _edge_body,
        grid=(E // be,),
        in_specs=[
            pl.BlockSpec((be, 128), lambda i: (i, 0)),
            pl.BlockSpec((be, 16), lambda i: (i, 0)),
            pl.BlockSpec((be, DE), lambda i: (i, 0)),
            wfull((1, 128)), wfull((DE, 128)), wfull((1, 128)),
            wfull((128, 128)), wfull((1, 128)),
            wfull((128, 128)), wfull((1, 128)),
            wfull((128, 1)), wfull((1, 1)),
        ],
        out_specs=[
            pl.BlockSpec((be, 128), lambda i: (i, 0)),
            pl.BlockSpec((be, 128), lambda i: (i, 0)),
        ],
        out_shape=[
            jax.ShapeDtypeStruct((E, 128), f32),
            jax.ShapeDtypeStruct((E, 128), f32),
        ],
    )(pre, dr, edge_attr, wr, wea, be1, we2, be2, wc1, bc1, wc2, bc2)


def _scatter_m_body(mij, row, z128, agg_out, idxv, mbuf, aggsh, semm0, semm1):
    c = lax.axis_index("c")
    s = lax.axis_index("s")
    wid = s * NC + c
    base = wid * PER_W
    rsl = pl.ds(s * ROWS_PER_TILE, ROWS_PER_TILE)
    pltpu.sync_copy(z128.at[rsl], aggsh.at[rsl])
    plsc.subcore_barrier()
    sems = (semm0, semm1)

    def start(k, p):
        b = base + k * C
        pltpu.sync_copy(row.at[pl.ds(b, C)], idxv.at[p])
        pltpu.async_copy(mij.at[pl.ds(b, C)], mbuf.at[p], sems[p])

    def body(p):
        pltpu.make_async_copy(mij.at[pl.ds(0, C)], mbuf.at[p],
                              sems[p]).wait()
        pltpu.sync_copy(mbuf.at[p], aggsh.at[idxv.at[p]], add=True)

    start(0, 0)
    start(1, 1)

    def chunk2(k2, carry):
        for p in range(2):
            k = k2 * 2 + p
            body(p)

            @pl.when(k + 2 < CH)
            def _():
                start(k + 2, p)
        return carry

    lax.fori_loop(0, CH // 2, chunk2, 0)

    @pl.when((CH % 2) == 1)
    def _():
        body(0)

    plsc.subcore_barrier()
    pltpu.sync_copy(aggsh.at[rsl], agg_out.at[c].at[rsl])


@functools.cache
def _scatter_m():
    return pl.kernel(
        _scatter_m_body,
        out_type=jax.ShapeDtypeStruct((NC, NPAD, 128), f32),
        mesh=plsc.VectorSubcoreMesh(core_axis_name="c", subcore_axis_name="s",
                                    num_cores=NC, num_subcores=NS),
        scratch_types=[
            pltpu.VMEM((2, C), i32),
            pltpu.VMEM((2, C, 128), f32),
            pltpu.VMEM_SHARED((NPAD, 128), f32),
            pltpu.SemaphoreType.DMA,
            pltpu.SemaphoreType.DMA,
        ],
    )


def _scatter_t_body(t8, row, z16, t_out, idxv, tbuf, tsh, semt0, semt1):
    c = lax.axis_index("c")
    s = lax.axis_index("s")
    wid = s * NC + c
    base = wid * PER_W
    rsl = pl.ds(s * ROWS_PER_TILE, ROWS_PER_TILE)
    pltpu.sync_copy(z16.at[rsl], tsh.at[rsl])
    plsc.subcore_barrier()
    sems = (semt0, semt1)

    def start(k, p):
        b = base + k * C
        pltpu.sync_copy(row.at[pl.ds(b, C)], idxv.at[p])
        pltpu.async_copy(t8.at[pl.ds(b, C)], tbuf.at[p], sems[p])

    def body(p):
        pltpu.make_async_copy(t8.at[pl.ds(0, C)], tbuf.at[p],
                              sems[p]).wait()
        pltpu.sync_copy(tbuf.at[p], tsh.at[idxv.at[p]], add=True)

    start(0, 0)
    start(1, 1)

    def chunk2(k2, carry):
        for p in range(2):
            k = k2 * 2 + p
            body(p)

            @pl.when(k + 2 < CH)
            def _():
                start(k + 2, p)
        return carry

    lax.fori_loop(0, CH // 2, chunk2, 0)

    @pl.when((CH % 2) == 1)
    def _():
        body(0)

    plsc.subcore_barrier()
    pltpu.sync_copy(tsh.at[rsl], t_out.at[c].at[rsl])


@functools.cache
def _scatter_t():
    return pl.kernel(
        _scatter_t_body,
        out_type=jax.ShapeDtypeStruct((NC, NPAD, 128), f32),
        mesh=plsc.VectorSubcoreMesh(core_axis_name="c", subcore_axis_name="s",
                                    num_cores=NC, num_subcores=NS),
        scratch_types=[
            pltpu.VMEM((2, C), i32),
            pltpu.VMEM((2, C, 128), f32),
            pltpu.VMEM_SHARED((NPAD, 128), f32),
            pltpu.SemaphoreType.DMA,
            pltpu.SemaphoreType.DMA,
        ],
    )


def _node_body(h_ref, a0_ref, a1_ref, t0_ref, t1_ref, coord_ref,
               wn1a_ref, wn1b_ref, bn1_ref, wn2_ref, bn2_ref,
               hout_ref, cout_ref):
    agg = a0_ref[0] + a1_ref[0]
    u = jnp.maximum(
        jnp.dot(h_ref[...], wn1a_ref[...], preferred_element_type=f32) +
        jnp.dot(agg, wn1b_ref[...], preferred_element_type=f32) +
        bn1_ref[...], 0.0)
    hout_ref[...] = (jnp.dot(u, wn2_ref[...], preferred_element_type=f32) +
                     bn2_ref[...])
    t = t0_ref[0] + t1_ref[0]
    s3 = t[:, 0:3]
    cnt = t[:, 3:4]
    cout_ref[...] = coord_ref[...] + s3 / jnp.maximum(cnt, 1.0)


def _node_model(h, agg_p, t_p, coord, wn1a, wn1b, bn1, wn2, bn2):
    bn = 1000
    wfull = lambda shape: pl.BlockSpec(shape, lambda i: (0, 0))
    return pl.pallas_call(
        _node_body,
        grid=(N // bn,),
        in_specs=[
            pl.BlockSpec((bn, 128), lambda i: (i, 0)),
            pl.BlockSpec((1, bn, 128), lambda i: (0, i, 0)),
            pl.BlockSpec((1, bn, 128), lambda i: (1, i, 0)),
            pl.BlockSpec((1, bn, 128), lambda i: (0, i, 0)),
            pl.BlockSpec((1, bn, 128), lambda i: (1, i, 0)),
            pl.BlockSpec((bn, 3), lambda i: (i, 0)),
            wfull((128, 128)), wfull((128, 128)), wfull((1, 128)),
            wfull((128, 128)), wfull((1, 128)),
        ],
        out_specs=[
            pl.BlockSpec((bn, 128), lambda i: (i, 0)),
            pl.BlockSpec((bn, 3), lambda i: (i, 0)),
        ],
        out_shape=[
            jax.ShapeDtypeStruct((N, 128), f32),
            jax.ShapeDtypeStruct((N, 3), f32),
        ],
    )(h, agg_p, agg_p, t_p, t_p, coord, wn1a, wn1b, bn1, wn2, bn2)


def kernel(h, edge_index, coord, edge_attr,
           W_e1, b_e1, W_e2, b_e2,
           W_n1, b_n1, W_n2, b_n2,
           W_c1, b_c1, W_c2, b_c2):
    row = edge_index[0]
    col = edge_index[1]
    c16 = jnp.pad(coord, ((0, 0), (0, 13)))
    w1a = W_e1[0:D]
    w1b = W_e1[D:2 * D]
    wr = W_e1[2 * D:2 * D + 1]
    wea = W_e1[2 * D + 1:]
    ta, tb = _prep_tables(h, w1a, w1b)
    pre = _gather_pre()(ta, tb, row, col)
    dr = _coord_feats()(c16, row, col)
    m_ij, t8 = _edge_mlp(pre, dr, edge_attr,
                         wr, wea, b_e1.reshape(1, H),
                         W_e2, b_e2.reshape(1, H),
                         W_c1, b_c1.reshape(1, H),
                         W_c2, b_c2.reshape(1, 1))
    z128 = jnp.zeros((NPAD, 128), f32)
    agg_p = _scatter_m()(m_ij, row, z128)
    t_p = _scatter_t()(t8, row, z128)
    h_out, coord_out = _node_model(h, agg_p, t_p, coord,
                                   W_n1[0:D], W_n1[D:], b_n1.reshape(1, H),
                                   W_n2, b_n2.reshape(1, H))
    return (h_out, coord_out, m_ij)
